# speedup vs baseline: 1.6764x; 1.6764x over previous
import jax
import jax.numpy as jnp
from jax.experimental import pallas as pl


def _normalize_body(h_ref, hn_ref):
    x = h_ref[...]
    n2 = jnp.sum(x * x, axis=1, keepdims=True)
    inv = 1.0 / jnp.maximum(jnp.sqrt(n2), 1e-12)
    hn_ref[...] = x * inv


def kernel(H, edge_index):
    N, D = H.shape
    BR = 1000
    Hn = pl.pallas_call(
        _normalize_body,
        out_shape=jax.ShapeDtypeStruct((N, D), jnp.float32),
        grid=(N // BR,),
        in_specs=[pl.BlockSpec((BR, D), lambda i: (i, 0))],
        out_specs=pl.BlockSpec((BR, D), lambda i: (i, 0)),
    )(H)

    row = edge_index[0]
    col = edge_index[1]
    e = jnp.sum(Hn[row] * Hn[col], axis=1)
    w = jnp.exp(e)
    s = jax.ops.segment_sum(w, row, num_segments=N)
    acc = jax.ops.segment_sum(w[:, None] * H[col], row, num_segments=N)
    return jnp.where(s[:, None] > 0, acc / jnp.maximum(s, 1e-30)[:, None], 0.0)


# speedup vs baseline: 2.4998x; 1.4912x over previous
import functools

import jax
import jax.numpy as jnp
from jax import lax
from jax.experimental import pallas as pl
from jax.experimental.pallas import tpu as pltpu
from jax.experimental.pallas import tpu_sc as plsc

N = 10000
D = 256
NPAD = 10240
C = 128
NC, NS = 2, 16
NW = NC * NS


def _shuffle(x, idx):
    return lax.gather(
        x, idx[:, None],
        dimension_numbers=lax.GatherDimensionNumbers(
            offset_dims=(), collapsed_slice_dims=(0,), start_index_map=(0,)),
        slice_sizes=(1,),
        mode=lax.GatherScatterMode.PROMISE_IN_BOUNDS)


def _norm_body(h_ref, hn_ref, hlo_ref, hhi_ref):
    x = h_ref[...]
    n2 = jnp.sum(x * x, axis=1, keepdims=True)
    inv = lax.rsqrt(jnp.maximum(n2, 1e-24))
    hn_ref[...] = x * inv
    hlo_ref[...] = x[:, : D // 2]
    hhi_ref[...] = x[:, D // 2:]


def _div_body(a0_ref, a1_ref, s0_ref, s1_ref, out_ref):
    st = s0_ref[...] + s1_ref[...]
    inv = jnp.where(st > 0.0, 1.0 / jnp.maximum(st, 1e-30), 0.0)
    out_ref[:, : D // 2] = a0_ref[...] * inv
    out_ref[:, D // 2:] = a1_ref[...] * inv


def _sc_pass1(epad, hn_hbm, row_hbm, col_hbm, w_hbm, s_hbm,
              idx_r, idx_c, buf_a, buf_b, wbuf, s_sh, sem):
    c = lax.axis_index("c")
    sid = lax.axis_index("s")
    wid = sid * NC + c
    epw = epad // NW
    nchunks = epw // C
    lane = lax.iota(jnp.int32, 16)

    def zero_w(i, _):
        wbuf[pl.ds(i * 16, 16)] = jnp.zeros((16,), jnp.float32)
        return 0
    lax.fori_loop(0, C // 16, zero_w, 0)
    for t in range(NPAD // NS // C):
        pltpu.sync_copy(wbuf, s_sh.at[pl.ds(sid * (NPAD // NS) + t * C, C)])
    plsc.subcore_barrier()

    def chunk(k, _):
        base = wid * epw + k * C
        pltpu.sync_copy(row_hbm.at[pl.ds(base, C)], idx_r)
        pltpu.sync_copy(col_hbm.at[pl.ds(base, C)], idx_c)
        pltpu.async_copy(hn_hbm.at[idx_r], buf_a, sem).wait()
        pltpu.async_copy(hn_hbm.at[idx_c], buf_b, sem).wait()

        def grp(g, _):
            dots = jnp.zeros((16,), jnp.float32)
            for l in range(16):
                j = g * 16 + l
                acc = buf_a[j, pl.ds(0, 16)] * buf_b[j, pl.ds(0, 16)]
                for v in range(1, 16):
                    acc = acc + buf_a[j, pl.ds(v * 16, 16)] * buf_b[j, pl.ds(v * 16, 16)]
                for k in (8, 4, 2, 1):
                    acc = acc + _shuffle(acc, jnp.bitwise_xor(lane, k))
                dots = jnp.where(lane == l, acc, dots)
            wv = jnp.exp(dots)
            gvec = (base + g * 16) + lane
            wv = jnp.where(gvec < E_EDGES, wv, jnp.zeros((16,), jnp.float32))
            wbuf[pl.ds(g * 16, 16)] = wv
            return 0

        lax.fori_loop(0, C // 16, grp, 0)
        pltpu.sync_copy(wbuf, s_sh.at[idx_r], add=True)
        pltpu.sync_copy(wbuf, w_hbm.at[pl.ds(base, C)])
        return 0

    lax.fori_loop(0, nchunks, chunk, 0)
    plsc.subcore_barrier()
    sp = NPAD // NS
    pltpu.sync_copy(s_sh.at[pl.ds(sid * sp, sp)], s_hbm.at[c, pl.ds(sid * sp, sp)])


def _sc_pass2(epad, hlo_hbm, hhi_hbm, row_hbm, col_hbm, w_hbm, acc_hbm,
              idx_r, idx_c, sbuf, wbuf, acc_sh, sem):
    c = lax.axis_index("c")
    sid = lax.axis_index("s")
    ept = epad // NS
    nchunks = ept // C
    hw = D // 2

    def zero_row(i, _):
        for v in range(hw // 16):
            sbuf[i, pl.ds(v * 16, 16)] = jnp.zeros((16,), jnp.float32)
        return 0
    lax.fori_loop(0, C, zero_row, 0)
    for t in range(NPAD // NS // C):
        pltpu.sync_copy(sbuf, acc_sh.at[pl.ds(sid * (NPAD // NS) + t * C, C)])
    plsc.subcore_barrier()

    def chunk(k, _):
        base = sid * ept + k * C
        pltpu.sync_copy(row_hbm.at[pl.ds(base, C)], idx_r)
        pltpu.sync_copy(col_hbm.at[pl.ds(base, C)], idx_c)
        pltpu.sync_copy(w_hbm.at[pl.ds(base, C)], wbuf)

        @pl.when(c == 0)
        def _():
            pltpu.async_copy(hlo_hbm.at[idx_c], sbuf, sem).wait()

        @pl.when(c == 1)
        def _():
            pltpu.async_copy(hhi_hbm.at[idx_c], sbuf, sem).wait()

        def grp(g, _):
            wv = wbuf[pl.ds(g * 16, 16)]
            for l in range(16):
                j = g * 16 + l
                ws = _shuffle(wv, jnp.full((16,), l, jnp.int32))
                for v in range(hw // 16):
                    sbuf[j, pl.ds(v * 16, 16)] = sbuf[j, pl.ds(v * 16, 16)] * ws
            return 0

        lax.fori_loop(0, C // 16, grp, 0)
        pltpu.sync_copy(sbuf, acc_sh.at[idx_r], add=True)
        return 0

    lax.fori_loop(0, nchunks, chunk, 0)
    plsc.subcore_barrier()
    sp = NPAD // NS
    pltpu.sync_copy(acc_sh.at[pl.ds(sid * sp, sp)], acc_hbm.at[c, pl.ds(sid * sp, sp)])


E_EDGES = 160000


def kernel(H, edge_index):
    epad = ((E_EDGES + NW * C - 1) // (NW * C)) * (NW * C)
    mesh = plsc.VectorSubcoreMesh(core_axis_name="c", subcore_axis_name="s")

    br = 1000
    Hn, Hlo, Hhi = pl.pallas_call(
        _norm_body,
        out_shape=[
            jax.ShapeDtypeStruct((N, D), jnp.float32),
            jax.ShapeDtypeStruct((N, D // 2), jnp.float32),
            jax.ShapeDtypeStruct((N, D // 2), jnp.float32),
        ],
        grid=(N // br,),
        in_specs=[pl.BlockSpec((br, D), lambda i: (i, 0))],
        out_specs=[
            pl.BlockSpec((br, D), lambda i: (i, 0)),
            pl.BlockSpec((br, D // 2), lambda i: (i, 0)),
            pl.BlockSpec((br, D // 2), lambda i: (i, 0)),
        ],
    )(H)

    pad = jnp.zeros((epad - E_EDGES,), jnp.int32)
    rowp = jnp.concatenate([edge_index[0], pad])
    colp = jnp.concatenate([edge_index[1], pad])

    sc_params = pltpu.CompilerParams(use_tc_tiling_on_sc=False)
    pass1 = pl.kernel(
        functools.partial(_sc_pass1, epad),
        compiler_params=sc_params,
        out_type=[
            jax.ShapeDtypeStruct((epad,), jnp.float32),
            jax.ShapeDtypeStruct((NC, NPAD), jnp.float32),
        ],
        mesh=mesh,
        scratch_types=[
            pltpu.VMEM((C,), jnp.int32),
            pltpu.VMEM((C,), jnp.int32),
            pltpu.VMEM((C, D), jnp.float32),
            pltpu.VMEM((C, D), jnp.float32),
            pltpu.VMEM((C,), jnp.float32),
            pltpu.VMEM_SHARED((NPAD,), jnp.float32),
            pltpu.SemaphoreType.DMA,
        ],
    )
    w, s = pass1(Hn, rowp, colp)

    pass2 = pl.kernel(
        functools.partial(_sc_pass2, epad),
        compiler_params=sc_params,
        out_type=jax.ShapeDtypeStruct((NC, NPAD, D // 2), jnp.float32),
        mesh=mesh,
        scratch_types=[
            pltpu.VMEM((C,), jnp.int32),
            pltpu.VMEM((C,), jnp.int32),
            pltpu.VMEM((C, D // 2), jnp.float32),
            pltpu.VMEM((C,), jnp.float32),
            pltpu.VMEM_SHARED((NPAD, D // 2), jnp.float32),
            pltpu.SemaphoreType.DMA,
        ],
    )
    acc = pass2(Hlo, Hhi, rowp, colp, w)

    out = pl.pallas_call(
        _div_body,
        out_shape=jax.ShapeDtypeStruct((N, D), jnp.float32),
        grid=(N // br,),
        in_specs=[
            pl.BlockSpec((br, D // 2), lambda i: (i, 0)),
            pl.BlockSpec((br, D // 2), lambda i: (i, 0)),
            pl.BlockSpec((br, 1), lambda i: (i, 0)),
            pl.BlockSpec((br, 1), lambda i: (i, 0)),
        ],
        out_specs=pl.BlockSpec((br, D), lambda i: (i, 0)),
    )(acc[0, :N], acc[1, :N], s[0, :N, None], s[1, :N, None])
    return out


# speedup vs baseline: 3.7044x; 1.4819x over previous
import functools

import jax
import jax.numpy as jnp
from jax import lax
from jax.experimental import pallas as pl
from jax.experimental.pallas import tpu as pltpu
from jax.experimental.pallas import tpu_sc as plsc

N = 10000
D = 256
NPAD = 10240
C = 64
C2 = 64
NC, NS = 2, 16
NW = NC * NS


def _shuffle(x, idx):
    return lax.gather(
        x, idx[:, None],
        dimension_numbers=lax.GatherDimensionNumbers(
            offset_dims=(), collapsed_slice_dims=(0,), start_index_map=(0,)),
        slice_sizes=(1,),
        mode=lax.GatherScatterMode.PROMISE_IN_BOUNDS)


def _norm_body(h_ref, hn_ref, hlo_ref, hhi_ref):
    x = h_ref[...]
    n2 = jnp.sum(x * x, axis=1, keepdims=True)
    inv = lax.rsqrt(jnp.maximum(n2, 1e-24))
    hn_ref[...] = x * inv
    hlo_ref[...] = x[:, : D // 2]
    hhi_ref[...] = x[:, D // 2:]


def _div_body(a0_ref, a1_ref, s0_ref, s1_ref, out_ref):
    st = s0_ref[...] + s1_ref[...]
    inv = jnp.where(st > 0.0, 1.0 / jnp.maximum(st, 1e-30), 0.0)
    out_ref[:, : D // 2] = a0_ref[...] * inv
    out_ref[:, D // 2:] = a1_ref[...] * inv


def _sc_pass1(epad, hn_hbm, row2d_hbm, col2d_hbm, w_hbm, s_hbm,
              idx_r, idx_c, bufs_a, bufs_b, wtile, s_sh,
              sa0, sa1, sb0, sb1):
    c = lax.axis_index("c")
    sid = lax.axis_index("s")
    wid = sid * NC + c
    epw = epad // NW
    nchunks = epw // C
    lane = lax.iota(jnp.int32, 16)
    sems_a = (sa0, sa1)
    sems_b = (sb0, sb1)

    pltpu.sync_copy(row2d_hbm.at[pl.ds(wid * nchunks, nchunks)], idx_r)
    pltpu.sync_copy(col2d_hbm.at[pl.ds(wid * nchunks, nchunks)], idx_c)

    def zero_w(i, _):
        wtile[pl.ds(i * 16, 16)] = jnp.zeros((16,), jnp.float32)
        return 0
    lax.fori_loop(0, epw // 16, zero_w, 0)
    sp = NPAD // NS
    for t in range(sp // C):
        pltpu.sync_copy(wtile.at[pl.ds(t * C, C)],
                        s_sh.at[pl.ds(sid * sp + t * C, C)])
    plsc.subcore_barrier()

    def issue(k, b):
        pltpu.async_copy(hn_hbm.at[idx_r.at[k]], bufs_a.at[b], sems_a[b])
        pltpu.async_copy(hn_hbm.at[idx_c.at[k]], bufs_b.at[b], sems_b[b])

    def wait(k, b):
        pltpu.make_async_copy(hn_hbm.at[idx_r.at[k]], bufs_a.at[b], sems_a[b]).wait()
        pltpu.make_async_copy(hn_hbm.at[idx_c.at[k]], bufs_b.at[b], sems_b[b]).wait()

    def compute(k, b):
        def grp(g, _):
            dots = jnp.zeros((16,), jnp.float32)
            for l in range(16):
                j = g * 16 + l
                acc = bufs_a[b, j, pl.ds(0, 16)] * bufs_b[b, j, pl.ds(0, 16)]
                for v in range(1, D // 16):
                    acc = acc + (bufs_a[b, j, pl.ds(v * 16, 16)] *
                                 bufs_b[b, j, pl.ds(v * 16, 16)])
                for st in (8, 4, 2, 1):
                    acc = acc + _shuffle(acc, jnp.bitwise_xor(lane, st))
                dots = jnp.where(lane == l, acc, dots)
            wv = jnp.exp(dots)
            gvec = (wid * epw + k * C + g * 16) + lane
            wv = jnp.where(gvec < E_EDGES, wv, jnp.zeros((16,), jnp.float32))
            wtile[pl.ds(k * C + g * 16, 16)] = wv
            return 0

        lax.fori_loop(0, C // 16, grp, 0)

    def drain(k, b):
        wait(k, b)
        compute(k, b)
        pltpu.sync_copy(wtile.at[pl.ds(k * C, C)],
                        s_sh.at[idx_r.at[k]], add=True)

    def chunk(k, _):
        issue(k, 0)
        drain(k, 0)
        return 0

    lax.fori_loop(0, nchunks, chunk, 0)
    pltpu.sync_copy(wtile, w_hbm.at[pl.ds(wid * epw, epw)])
    plsc.subcore_barrier()
    pltpu.sync_copy(s_sh.at[pl.ds(sid * sp, sp)], s_hbm.at[c, pl.ds(sid * sp, sp)])


def _sc_pass2(epad, hlo_hbm, hhi_hbm, row2d_hbm, col2d_hbm, w_hbm, acc_hbm,
              idx_r, idx_c, sbufs, wtile, acc_sh, se0, se1):
    c = lax.axis_index("c")
    sid = lax.axis_index("s")
    ept = epad // NS
    nchunks = ept // C2
    hw = D // 2
    sems = (se0, se1)

    pltpu.sync_copy(row2d_hbm.at[pl.ds(sid * nchunks, nchunks)], idx_r)
    pltpu.sync_copy(col2d_hbm.at[pl.ds(sid * nchunks, nchunks)], idx_c)
    pltpu.sync_copy(w_hbm.at[pl.ds(sid * ept, ept)], wtile)

    def zero_row(i, _):
        for v in range(hw // 16):
            sbufs[0, i, pl.ds(v * 16, 16)] = jnp.zeros((16,), jnp.float32)
        return 0
    lax.fori_loop(0, C2, zero_row, 0)
    sp = NPAD // NS
    for t in range(sp // C2):
        pltpu.sync_copy(sbufs.at[0], acc_sh.at[pl.ds(sid * sp + t * C2, C2)])
    plsc.subcore_barrier()

    def issue(k, b):
        @pl.when(c == 0)
        def _():
            pltpu.async_copy(hlo_hbm.at[idx_c.at[k]], sbufs.at[b], sems[b])

        @pl.when(c == 1)
        def _():
            pltpu.async_copy(hhi_hbm.at[idx_c.at[k]], sbufs.at[b], sems[b])

    def wait(k, b):
        @pl.when(c == 0)
        def _():
            pltpu.make_async_copy(hlo_hbm.at[idx_c.at[k]], sbufs.at[b], sems[b]).wait()

        @pl.when(c == 1)
        def _():
            pltpu.make_async_copy(hhi_hbm.at[idx_c.at[k]], sbufs.at[b], sems[b]).wait()

    def compute(k, b):
        def grp(g, _):
            wv = wtile[pl.ds(k * C2 + g * 16, 16)]
            for l in range(16):
                j = g * 16 + l
                ws = _shuffle(wv, jnp.full((16,), l, jnp.int32))
                for v in range(hw // 16):
                    sbufs[b, j, pl.ds(v * 16, 16)] = sbufs[b, j, pl.ds(v * 16, 16)] * ws
            return 0

        lax.fori_loop(0, C2 // 16, grp, 0)

    def drain(k, b):
        wait(k, b)
        compute(k, b)
        pltpu.sync_copy(sbufs.at[b], acc_sh.at[idx_r.at[k]], add=True)

    issue(0, 0)
    issue(1, 1)

    def pair(p, _):
        for b in range(2):
            k = p * 2 + b
            drain(k, b)
            issue(k + 2, b)
        return 0

    lax.fori_loop(0, nchunks // 2 - 1, pair, 0)
    for b in range(2):
        drain(nchunks - 2 + b, b)
    plsc.subcore_barrier()
    pltpu.sync_copy(acc_sh.at[pl.ds(sid * sp, sp)], acc_hbm.at[c, pl.ds(sid * sp, sp)])


E_EDGES = 160000


def kernel(H, edge_index):
    epad = ((E_EDGES + NW * C - 1) // (NW * C)) * (NW * C)
    mesh = plsc.VectorSubcoreMesh(core_axis_name="c", subcore_axis_name="s")

    br = 1000
    Hn, Hlo, Hhi = pl.pallas_call(
        _norm_body,
        out_shape=[
            jax.ShapeDtypeStruct((N, D), jnp.float32),
            jax.ShapeDtypeStruct((N, D // 2), jnp.float32),
            jax.ShapeDtypeStruct((N, D // 2), jnp.float32),
        ],
        grid=(N // br,),
        in_specs=[pl.BlockSpec((br, D), lambda i: (i, 0))],
        out_specs=[
            pl.BlockSpec((br, D), lambda i: (i, 0)),
            pl.BlockSpec((br, D // 2), lambda i: (i, 0)),
            pl.BlockSpec((br, D // 2), lambda i: (i, 0)),
        ],
    )(H)

    pad = jnp.zeros((epad - E_EDGES,), jnp.int32)
    rowp = jnp.concatenate([edge_index[0], pad])
    colp = jnp.concatenate([edge_index[1], pad])
    row2d_1 = rowp.reshape(epad // C, C)
    col2d_1 = colp.reshape(epad // C, C)
    row2d_2 = rowp.reshape(epad // C2, C2)
    col2d_2 = colp.reshape(epad // C2, C2)
    epw = epad // NW

    sc_params = pltpu.CompilerParams(use_tc_tiling_on_sc=False)
    pass1 = pl.kernel(
        functools.partial(_sc_pass1, epad),
        compiler_params=sc_params,
        out_type=[
            jax.ShapeDtypeStruct((epad,), jnp.float32),
            jax.ShapeDtypeStruct((NC, NPAD), jnp.float32),
        ],
        mesh=mesh,
        scratch_types=[
            pltpu.VMEM((epw // C, C), jnp.int32),
            pltpu.VMEM((epw // C, C), jnp.int32),
            pltpu.VMEM((2, C, D), jnp.float32),
            pltpu.VMEM((2, C, D), jnp.float32),
            pltpu.VMEM((epw,), jnp.float32),
            pltpu.VMEM_SHARED((NPAD,), jnp.float32),
            pltpu.SemaphoreType.DMA,
            pltpu.SemaphoreType.DMA,
            pltpu.SemaphoreType.DMA,
            pltpu.SemaphoreType.DMA,
        ],
    )
    w, s = pass1(Hn, row2d_1, col2d_1)

    ept = epad // NS
    pass2 = pl.kernel(
        functools.partial(_sc_pass2, epad),
        compiler_params=sc_params,
        out_type=jax.ShapeDtypeStruct((NC, NPAD, D // 2), jnp.float32),
        mesh=mesh,
        scratch_types=[
            pltpu.VMEM((ept // C2, C2), jnp.int32),
            pltpu.VMEM((ept // C2, C2), jnp.int32),
            pltpu.VMEM((2, C2, D // 2), jnp.float32),
            pltpu.VMEM((ept,), jnp.float32),
            pltpu.VMEM_SHARED((NPAD, D // 2), jnp.float32),
            pltpu.SemaphoreType.DMA,
            pltpu.SemaphoreType.DMA,
        ],
    )
    acc = pass2(Hlo, Hhi, row2d_2, col2d_2, w)

    out = pl.pallas_call(
        _div_body,
        out_shape=jax.ShapeDtypeStruct((N, D), jnp.float32),
        grid=(N // br,),
        in_specs=[
            pl.BlockSpec((br, D // 2), lambda i: (i, 0)),
            pl.BlockSpec((br, D // 2), lambda i: (i, 0)),
            pl.BlockSpec((br, 1), lambda i: (i, 0)),
            pl.BlockSpec((br, 1), lambda i: (i, 0)),
        ],
        out_specs=pl.BlockSpec((br, D), lambda i: (i, 0)),
    )(acc[0, :N], acc[1, :N], s[0, :N, None], s[1, :N, None])
    return out


# speedup vs baseline: 4.1702x; 1.1257x over previous
import functools

import jax
import jax.numpy as jnp
from jax import lax
from jax.experimental import pallas as pl
from jax.experimental.pallas import tpu as pltpu
from jax.experimental.pallas import tpu_sc as plsc

N = 10000
D = 256
NPAD = 10240
C = 64
SC1 = 64
C2 = 64
NC, NS = 2, 16
NW = NC * NS


def _shuffle(x, idx):
    return lax.gather(
        x, idx[:, None],
        dimension_numbers=lax.GatherDimensionNumbers(
            offset_dims=(), collapsed_slice_dims=(0,), start_index_map=(0,)),
        slice_sizes=(1,),
        mode=lax.GatherScatterMode.PROMISE_IN_BOUNDS)


def _norm_body(h_ref, hn_ref, hlo_ref, hhi_ref):
    x = h_ref[...]
    n2 = jnp.sum(x * x, axis=1, keepdims=True)
    inv = lax.rsqrt(jnp.maximum(n2, 1e-24))
    hn_ref[...] = x * inv
    hlo_ref[...] = x[:, : D // 2]
    hhi_ref[...] = x[:, D // 2:]


def _div_body(a0_ref, a1_ref, s0_ref, s1_ref, out_ref):
    st = s0_ref[...] + s1_ref[...]
    inv = jnp.where(st > 0.0, 1.0 / jnp.maximum(st, 1e-30), 0.0)
    out_ref[:, : D // 2] = a0_ref[...] * inv
    out_ref[:, D // 2:] = a1_ref[...] * inv


def _sc_pass1(epad, hn_hbm, row2d_hbm, col2d_hbm, w_hbm, s_hbm,
              idx_r, idx_c, bufs_a, bufs_b, wtile, s_sh,
              sa0, sa1, sb0, sb1):
    c = lax.axis_index("c")
    sid = lax.axis_index("s")
    wid = sid * NC + c
    epw = epad // NW
    nchunks = epw // C
    lane = lax.iota(jnp.int32, 16)
    sems_a = (sa0, sa1)
    sems_b = (sb0, sb1)

    nscat = epw // SC1
    pltpu.sync_copy(row2d_hbm.at[pl.ds(wid * nchunks, nchunks)], idx_r)
    pltpu.sync_copy(col2d_hbm.at[pl.ds(wid * nchunks, nchunks)], idx_c)

    def zero_w(i, _):
        wtile[pl.ds(i * 16, 16)] = jnp.zeros((16,), jnp.float32)
        return 0
    lax.fori_loop(0, epw // 16, zero_w, 0)
    sp = NPAD // NS
    for t in range(sp // SC1):
        pltpu.sync_copy(wtile.at[pl.ds(t * SC1, SC1)],
                        s_sh.at[pl.ds(sid * sp + t * SC1, SC1)])
    plsc.subcore_barrier()

    def compute(k, b):
        def grp(g, _):
            dots = jnp.zeros((16,), jnp.float32)
            for l in range(16):
                j = g * 16 + l
                a0 = bufs_a[b, j, pl.ds(0, 16)] * bufs_b[b, j, pl.ds(0, 16)]
                a1 = bufs_a[b, j, pl.ds(16, 16)] * bufs_b[b, j, pl.ds(16, 16)]
                a2 = bufs_a[b, j, pl.ds(32, 16)] * bufs_b[b, j, pl.ds(32, 16)]
                a3 = bufs_a[b, j, pl.ds(48, 16)] * bufs_b[b, j, pl.ds(48, 16)]
                for v in range(4, D // 16, 4):
                    a0 = a0 + (bufs_a[b, j, pl.ds(v * 16, 16)] *
                               bufs_b[b, j, pl.ds(v * 16, 16)])
                    a1 = a1 + (bufs_a[b, j, pl.ds(v * 16 + 16, 16)] *
                               bufs_b[b, j, pl.ds(v * 16 + 16, 16)])
                    a2 = a2 + (bufs_a[b, j, pl.ds(v * 16 + 32, 16)] *
                               bufs_b[b, j, pl.ds(v * 16 + 32, 16)])
                    a3 = a3 + (bufs_a[b, j, pl.ds(v * 16 + 48, 16)] *
                               bufs_b[b, j, pl.ds(v * 16 + 48, 16)])
                acc = (a0 + a1) + (a2 + a3)
                for st in (8, 4, 2, 1):
                    acc = acc + _shuffle(acc, jnp.bitwise_xor(lane, st))
                dots = jnp.where(lane == l, acc, dots)
            wv = jnp.exp(dots)
            gvec = (wid * epw + k * C + g * 16) + lane
            wv = jnp.where(gvec < E_EDGES, wv, jnp.zeros((16,), jnp.float32))
            wtile[pl.ds(k * C + g * 16, 16)] = wv
            return 0

        lax.fori_loop(0, C // 16, grp, 0)

    def chunk(k, _):
        da = pltpu.async_copy(hn_hbm.at[idx_r.at[k]], bufs_a.at[0], sems_a[0])
        db = pltpu.async_copy(hn_hbm.at[idx_c.at[k]], bufs_b.at[0], sems_b[0])
        da.wait()
        db.wait()
        compute(k, 0)
        pltpu.sync_copy(wtile.at[pl.ds(k * C, C)],
                        s_sh.at[idx_r.at[k]], add=True)
        return 0

    lax.fori_loop(0, nchunks, chunk, 0)
    pltpu.sync_copy(wtile, w_hbm.at[pl.ds(wid * epw, epw)])
    plsc.subcore_barrier()
    pltpu.sync_copy(s_sh.at[pl.ds(sid * sp, sp)], s_hbm.at[c, pl.ds(sid * sp, sp)])


def _sc_pass2(epad, hlo_hbm, hhi_hbm, row2d_hbm, col2d_hbm, w_hbm, acc_hbm,
              idx_r, idx_c, sbufs, wtile, acc_sh, se0, se1):
    c = lax.axis_index("c")
    sid = lax.axis_index("s")
    ept = epad // NS
    nchunks = ept // C2
    hw = D // 2
    sems = (se0, se1)

    pltpu.sync_copy(row2d_hbm.at[pl.ds(sid * nchunks, nchunks)], idx_r)
    pltpu.sync_copy(col2d_hbm.at[pl.ds(sid * nchunks, nchunks)], idx_c)
    pltpu.sync_copy(w_hbm.at[pl.ds(sid * ept, ept)], wtile)

    def zero_row(i, _):
        for v in range(hw // 16):
            sbufs[0, i, pl.ds(v * 16, 16)] = jnp.zeros((16,), jnp.float32)
        return 0
    lax.fori_loop(0, C2, zero_row, 0)
    sp = NPAD // NS
    for t in range(sp // C2):
        pltpu.sync_copy(sbufs.at[0], acc_sh.at[pl.ds(sid * sp + t * C2, C2)])
    plsc.subcore_barrier()

    def issue(k, b):
        @pl.when(c == 0)
        def _():
            pltpu.async_copy(hlo_hbm.at[idx_c.at[k]], sbufs.at[b], sems[b])

        @pl.when(c == 1)
        def _():
            pltpu.async_copy(hhi_hbm.at[idx_c.at[k]], sbufs.at[b], sems[b])

    def wait(k, b):
        @pl.when(c == 0)
        def _():
            pltpu.make_async_copy(hlo_hbm.at[idx_c.at[k]], sbufs.at[b], sems[b]).wait()

        @pl.when(c == 1)
        def _():
            pltpu.make_async_copy(hhi_hbm.at[idx_c.at[k]], sbufs.at[b], sems[b]).wait()

    def compute(k, b):
        def grp(g, _):
            wv = wtile[pl.ds(k * C2 + g * 16, 16)]
            for l in range(16):
                j = g * 16 + l
                ws = _shuffle(wv, jnp.full((16,), l, jnp.int32))
                for v in range(hw // 16):
                    sbufs[b, j, pl.ds(v * 16, 16)] = sbufs[b, j, pl.ds(v * 16, 16)] * ws
            return 0

        lax.fori_loop(0, C2 // 16, grp, 0)

    def drain(k, b):
        wait(k, b)
        compute(k, b)
        pltpu.sync_copy(sbufs.at[b], acc_sh.at[idx_r.at[k]], add=True)

    issue(0, 0)
    issue(1, 1)

    def pair(p, _):
        for b in range(2):
            k = p * 2 + b
            drain(k, b)
            issue(k + 2, b)
        return 0

    lax.fori_loop(0, nchunks // 2 - 1, pair, 0)
    for b in range(2):
        drain(nchunks - 2 + b, b)
    plsc.subcore_barrier()
    pltpu.sync_copy(acc_sh.at[pl.ds(sid * sp, sp)], acc_hbm.at[c, pl.ds(sid * sp, sp)])


E_EDGES = 160000


def kernel(H, edge_index):
    epad = ((E_EDGES + NW * C - 1) // (NW * C)) * (NW * C)
    mesh = plsc.VectorSubcoreMesh(core_axis_name="c", subcore_axis_name="s")

    br = 1000
    Hn, Hlo, Hhi = pl.pallas_call(
        _norm_body,
        out_shape=[
            jax.ShapeDtypeStruct((N, D), jnp.float32),
            jax.ShapeDtypeStruct((N, D // 2), jnp.float32),
            jax.ShapeDtypeStruct((N, D // 2), jnp.float32),
        ],
        grid=(N // br,),
        in_specs=[pl.BlockSpec((br, D), lambda i: (i, 0))],
        out_specs=[
            pl.BlockSpec((br, D), lambda i: (i, 0)),
            pl.BlockSpec((br, D // 2), lambda i: (i, 0)),
            pl.BlockSpec((br, D // 2), lambda i: (i, 0)),
        ],
    )(H)

    pad = jnp.zeros((epad - E_EDGES,), jnp.int32)
    rowp = jnp.concatenate([edge_index[0], pad])
    colp = jnp.concatenate([edge_index[1], pad])
    row2d_1 = rowp.reshape(epad // C, C)
    col2d_1 = colp.reshape(epad // C, C)
    row2d_2 = rowp.reshape(epad // C2, C2)
    col2d_2 = colp.reshape(epad // C2, C2)
    epw = epad // NW

    sc_params = pltpu.CompilerParams(use_tc_tiling_on_sc=False)
    pass1 = pl.kernel(
        functools.partial(_sc_pass1, epad),
        compiler_params=sc_params,
        out_type=[
            jax.ShapeDtypeStruct((epad,), jnp.float32),
            jax.ShapeDtypeStruct((NC, NPAD), jnp.float32),
        ],
        mesh=mesh,
        scratch_types=[
            pltpu.VMEM((epw // C, C), jnp.int32),
            pltpu.VMEM((epw // C, C), jnp.int32),
            pltpu.VMEM((2, C, D), jnp.float32),
            pltpu.VMEM((2, C, D), jnp.float32),
            pltpu.VMEM((epw,), jnp.float32),
            pltpu.VMEM_SHARED((NPAD,), jnp.float32),
            pltpu.SemaphoreType.DMA,
            pltpu.SemaphoreType.DMA,
            pltpu.SemaphoreType.DMA,
            pltpu.SemaphoreType.DMA,
        ],
    )
    w, s = pass1(Hn, row2d_1, col2d_1)

    ept = epad // NS
    pass2 = pl.kernel(
        functools.partial(_sc_pass2, epad),
        compiler_params=sc_params,
        out_type=jax.ShapeDtypeStruct((NC, NPAD, D // 2), jnp.float32),
        mesh=mesh,
        scratch_types=[
            pltpu.VMEM((ept // C2, C2), jnp.int32),
            pltpu.VMEM((ept // C2, C2), jnp.int32),
            pltpu.VMEM((2, C2, D // 2), jnp.float32),
            pltpu.VMEM((ept,), jnp.float32),
            pltpu.VMEM_SHARED((NPAD, D // 2), jnp.float32),
            pltpu.SemaphoreType.DMA,
            pltpu.SemaphoreType.DMA,
        ],
    )
    acc = pass2(Hlo, Hhi, row2d_2, col2d_2, w)

    out = pl.pallas_call(
        _div_body,
        out_shape=jax.ShapeDtypeStruct((N, D), jnp.float32),
        grid=(N // br,),
        in_specs=[
            pl.BlockSpec((br, D // 2), lambda i: (i, 0)),
            pl.BlockSpec((br, D // 2), lambda i: (i, 0)),
            pl.BlockSpec((br, 1), lambda i: (i, 0)),
            pl.BlockSpec((br, 1), lambda i: (i, 0)),
        ],
        out_specs=pl.BlockSpec((br, D), lambda i: (i, 0)),
    )(acc[0, :N], acc[1, :N], s[0, :N, None], s[1, :N, None])
    return out
